# baseline (speedup 1.0000x reference)
import functools

import jax
import jax.numpy as jnp
from jax import lax
from jax.experimental import pallas as pl
from jax.experimental.pallas import tpu as pltpu


def _fused_kernel(alpha_ref, x_ref, bm_ref, p_ref,
                  w1_ref, asd1_ref, b1_ref,
                  w2_ref, asd2_ref, b2_ref,
                  wih0_ref, whh0_ref, bih0_ref, bhh0_ref,
                  wih1_ref, whh1_ref, bih1_ref, bhh1_ref,
                  wm_ref, cb_ref, w1c_ref, b1c_ref, w2c_ref, b2c_ref,
                  out_ref, *, t_len, n_nodes, hidden, k_batch):
    a = alpha_ref[0, 0]
    n = n_nodes
    gn = t_len * n
    npair = t_len // 2

    sel2 = (lax.broadcasted_iota(jnp.int32, (2, 2 * n), 1) // n
            == lax.broadcasted_iota(jnp.int32, (2, 2 * n), 0)
            ).astype(jnp.float32)
    ones_col = jnp.ones((n, 1), jnp.float32)

    def gat_layer(h_in, w, asd_w, bias):
        h = jnp.dot(h_in, w, preferred_element_type=jnp.float32)
        asd = jnp.dot(h, asd_w, preferred_element_type=jnp.float32)
        a_dT = jnp.transpose(asd)
        outs = []
        for j in range(k_batch):
            for p in range(npair):
                base = j * gn + p * 2 * n
                a_s3 = jnp.concatenate(
                    [asd[base:base + n, 0:1],
                     asd[base + n:base + 2 * n, 0:1], ones_col], axis=1)
                m3 = jnp.concatenate(
                    [sel2, a_dT[1:2, base:base + 2 * n]], axis=0)
                e = jnp.dot(a_s3, m3,
                            preferred_element_type=jnp.float32)
                e = jnp.where(e > 0, e, 0.2 * e)
                e = e + bm_ref[j, p]
                m = jnp.max(e, axis=0, keepdims=True)
                pr = jnp.exp(e - m)
                denom = jnp.sum(pr, axis=0, keepdims=True)
                att = pr * pl.reciprocal(denom, approx=True)
                outs.append(lax.dot_general(
                    att[:, :n], h[base:base + n],
                    (((0,), (0,)), ((), ())),
                    preferred_element_type=jnp.float32))
                outs.append(lax.dot_general(
                    att[:, n:], h[base + n:base + 2 * n],
                    (((0,), (0,)), ((), ())),
                    preferred_element_type=jnp.float32))
        o = jnp.concatenate(outs, axis=0) + bias
        return jnp.where(o > 0, o, a * o)

    x = x_ref[...].reshape(k_batch * gn, -1)
    h1 = gat_layer(x, w1_ref[...], asd1_ref[...], b1_ref[...])
    h2 = gat_layer(h1, w2_ref[...], asd2_ref[...], b2_ref[...])

    nseq = k_batch * n
    h2p = jnp.concatenate(
        [jnp.dot(p_ref[...], h2[j * gn:(j + 1) * gn],
                 preferred_element_type=jnp.float32)
         for j in range(k_batch)], axis=0)
    gi0 = jnp.dot(h2p, wih0_ref[...],
                  preferred_element_type=jnp.float32) + bih0_ref[...]

    whh0 = whh0_ref[...]; bhh0 = bhh0_ref[...]
    wih1 = wih1_ref[...]; bih1 = bih1_ref[...]
    whh1 = whh1_ref[...]; bhh1 = bhh1_ref[...]

    h0 = jnp.zeros((nseq, hidden), jnp.float32)
    h1s = jnp.zeros((nseq, hidden), jnp.float32)
    xs = []
    for t in range(t_len):
        gi = jnp.concatenate(
            [gi0[j * gn + t * n:j * gn + (t + 1) * n]
             for j in range(k_batch)], axis=0)
        gh = jnp.dot(h0, whh0, preferred_element_type=jnp.float32) + bhh0
        rz = jax.nn.sigmoid(gi[:, :2 * hidden] + gh[:, :2 * hidden])
        r = rz[:, :hidden]
        z = rz[:, hidden:]
        ng = jnp.tanh(gi[:, 2 * hidden:] + r * gh[:, 2 * hidden:])
        h0 = ng + z * (h0 - ng)
        gi1 = jnp.dot(h0, wih1, preferred_element_type=jnp.float32) + bih1
        gh1 = jnp.dot(h1s, whh1, preferred_element_type=jnp.float32) + bhh1
        rz1 = jax.nn.sigmoid(gi1[:, :2 * hidden] + gh1[:, :2 * hidden])
        r1 = rz1[:, :hidden]
        z1 = rz1[:, hidden:]
        ng1 = jnp.tanh(gi1[:, 2 * hidden:] + r1 * gh1[:, 2 * hidden:])
        h1s = ng1 + z1 * (h1s - ng1)
        xs.append(h1s)

    xr = jnp.concatenate(xs, axis=1)
    rid = lax.broadcasted_iota(jnp.int32, (nseq, 1), 0) % n
    zrow = jnp.zeros((1, t_len * hidden), jnp.float32)
    pdn = jnp.where(rid == 0, 0.0,
                    jnp.concatenate([zrow, xr[:-1]], axis=0))
    pup = jnp.where(rid == n - 1, 0.0,
                    jnp.concatenate([xr[1:], zrow], axis=0))
    patches = jnp.concatenate([pdn, xr, pup], axis=1)

    conv = jnp.dot(patches, wm_ref[...],
                   preferred_element_type=jnp.float32) + cb_ref[...]
    conv = jnp.where(conv > 0, conv, a * conv)
    h = jnp.dot(conv, w1c_ref[...],
                preferred_element_type=jnp.float32) + b1c_ref[...]
    h = jnp.where(h > 0, h, a * h)
    res = jnp.dot(h, w2c_ref[...],
                  preferred_element_type=jnp.float32) + b2c_ref[...]
    out_ref[...] = res.reshape(k_batch, n, -1)


def kernel(x, adj, gat1_w, gat1_asrc, gat1_adst, gat1_bias,
           gat2_w, gat2_asrc, gat2_adst, gat2_bias, prelu_a,
           gru_wih0_t, gru_whh0_t, gru_bih0, gru_bhh0,
           gru_wih1_t, gru_whh1_t, gru_bih1, gru_bhh1,
           conv_w, conv_b, out1_w_t, out1_b, out2_w_t, out2_b):
    b, t, n, fin = x.shape
    gn = t * n
    hidden = gat2_w.shape[1]
    num_heads = gat1_w.shape[1] // hidden
    pred = conv_w.shape[0]
    out_f = out2_w_t.shape[1]

    x_flat = x.reshape(b, gn, fin)

    eye_n = jnp.eye(n, dtype=jnp.float32)
    allow = jnp.maximum(adj, eye_n)
    bm = jnp.where(allow > 0, 0.0, -1e30).astype(jnp.float32)
    bmp = bm.reshape(b, t // 2, 2, n, n).transpose(0, 1, 3, 2, 4)
    bmp = bmp.reshape(b, t // 2, n, 2 * n)

    asd1 = jnp.concatenate([gat1_asrc, gat1_adst], axis=0).T
    asd2 = jnp.concatenate([gat2_asrc, gat2_adst], axis=0).T

    rn = jnp.arange(gn)
    p512 = jnp.eye(gn, dtype=jnp.float32)[(rn % n) * t + rn // n]

    hh_idx = jnp.arange(hidden)
    band = hh_idx[:, None] - hh_idx[None, :]
    sel = jnp.stack([(band == dx - 1).astype(jnp.float32)
                     for dx in range(3)])
    wm = jnp.einsum('cdye,eab->ydacb', conv_w, sel).reshape(
        3 * t * hidden, pred * hidden)
    cb = jnp.repeat(conv_b[:, 0], hidden)[None, :]

    eye_c = jnp.eye(pred, dtype=jnp.float32)
    w1c = jnp.kron(eye_c, out1_w_t)
    b1c = jnp.tile(out1_b, (1, pred))
    w2c = jnp.kron(eye_c, out2_w_t)
    b2c = jnp.tile(out2_b, (1, pred))

    k_batch = 16
    kern = functools.partial(_fused_kernel, t_len=t, n_nodes=n,
                             hidden=hidden, k_batch=k_batch)
    rep = lambda i: (0, 0)
    out = pl.pallas_call(
        kern,
        out_shape=jax.ShapeDtypeStruct((b, n, pred * out_f), jnp.float32),
        grid=(b // k_batch,),
        in_specs=[
            pl.BlockSpec(memory_space=pltpu.MemorySpace.SMEM),
            pl.BlockSpec((k_batch, gn, fin), lambda i: (i, 0, 0)),
            pl.BlockSpec((k_batch, t // 2, n, 2 * n),
                         lambda i: (i, 0, 0, 0)),
            pl.BlockSpec((gn, gn), rep),
            pl.BlockSpec((fin, num_heads * hidden), rep),
            pl.BlockSpec((num_heads * hidden, 2), rep),
            pl.BlockSpec((1, num_heads * hidden), rep),
            pl.BlockSpec((num_heads * hidden, hidden), rep),
            pl.BlockSpec((hidden, 2), rep),
            pl.BlockSpec((1, hidden), rep),
            pl.BlockSpec((hidden, 3 * hidden), rep),
            pl.BlockSpec((hidden, 3 * hidden), rep),
            pl.BlockSpec((1, 3 * hidden), rep),
            pl.BlockSpec((1, 3 * hidden), rep),
            pl.BlockSpec((hidden, 3 * hidden), rep),
            pl.BlockSpec((hidden, 3 * hidden), rep),
            pl.BlockSpec((1, 3 * hidden), rep),
            pl.BlockSpec((1, 3 * hidden), rep),
            pl.BlockSpec((3 * t * hidden, pred * hidden), rep),
            pl.BlockSpec((1, pred * hidden), rep),
            pl.BlockSpec((pred * hidden, pred * hidden), rep),
            pl.BlockSpec((1, pred * hidden), rep),
            pl.BlockSpec((pred * hidden, pred * out_f), rep),
            pl.BlockSpec((1, pred * out_f), rep),
        ],
        out_specs=pl.BlockSpec((k_batch, n, pred * out_f),
                               lambda i: (i, 0, 0)),
        compiler_params=pltpu.CompilerParams(
            dimension_semantics=("parallel",)),
    )(prelu_a, x_flat, bmp, p512,
      gat1_w, asd1, gat1_bias,
      gat2_w, asd2, gat2_bias,
      gru_wih0_t, gru_whh0_t, gru_bih0, gru_bhh0,
      gru_wih1_t, gru_whh1_t, gru_bih1, gru_bhh1,
      wm, cb, w1c, b1c, w2c, b2c)

    return out.reshape(b, n, pred, out_f).transpose(0, 2, 1, 3)


# speedup vs baseline: 1.0245x; 1.0245x over previous
import functools

import jax
import jax.numpy as jnp
from jax import lax
from jax.experimental import pallas as pl
from jax.experimental.pallas import tpu as pltpu


def _fused_kernel(alpha_ref, x_ref, bm_ref, p_ref,
                  w1_ref, asd1_ref, b1_ref,
                  w2_ref, asd2_ref, b2_ref,
                  wih0_ref, bih0_ref, wg_ref,
                  wm_ref, cb_ref, w1c_ref, b1c_ref, w2c_ref, b2c_ref,
                  out_ref, *, t_len, n_nodes, hidden, k_batch):
    a = alpha_ref[0, 0]
    n = n_nodes
    gn = t_len * n
    npair = t_len // 2

    sel2 = (lax.broadcasted_iota(jnp.int32, (2, 2 * n), 1) // n
            == lax.broadcasted_iota(jnp.int32, (2, 2 * n), 0)
            ).astype(jnp.float32)
    ones_col = jnp.ones((n, 1), jnp.float32)

    def gat_layer(h_in, w, asd_w, bias):
        h = jnp.dot(h_in, w, preferred_element_type=jnp.float32)
        asd = jnp.dot(h, asd_w, preferred_element_type=jnp.float32)
        a_dT = jnp.transpose(asd)
        outs = []
        for j in range(k_batch):
            for p in range(npair):
                base = j * gn + p * 2 * n
                a_s3 = jnp.concatenate(
                    [asd[base:base + n, 0:1],
                     asd[base + n:base + 2 * n, 0:1], ones_col], axis=1)
                m3 = jnp.concatenate(
                    [sel2, a_dT[1:2, base:base + 2 * n]], axis=0)
                e = jnp.dot(a_s3, m3,
                            preferred_element_type=jnp.float32)
                e = jnp.where(e > 0, e, 0.2 * e)
                e = e + bm_ref[j, p]
                m = jnp.max(e, axis=0, keepdims=True)
                pr = jnp.exp(e - m)
                denom = jnp.sum(pr, axis=0, keepdims=True)
                att = pr * pl.reciprocal(denom, approx=True)
                outs.append(lax.dot_general(
                    att[:, :n], h[base:base + n],
                    (((0,), (0,)), ((), ())),
                    preferred_element_type=jnp.float32))
                outs.append(lax.dot_general(
                    att[:, n:], h[base + n:base + 2 * n],
                    (((0,), (0,)), ((), ())),
                    preferred_element_type=jnp.float32))
        o = jnp.concatenate(outs, axis=0) + bias
        return jnp.where(o > 0, o, a * o)

    x = x_ref[...].reshape(k_batch * gn, -1)
    h1 = gat_layer(x, w1_ref[...], asd1_ref[...], b1_ref[...])
    h2 = gat_layer(h1, w2_ref[...], asd2_ref[...], b2_ref[...])

    h2p = jnp.concatenate(
        [jnp.dot(p_ref[...], h2[j * gn:(j + 1) * gn],
                 preferred_element_type=jnp.float32)
         for j in range(k_batch)], axis=0)
    gi0 = jnp.dot(h2p, wih0_ref[...],
                  preferred_element_type=jnp.float32) + bih0_ref[...]

    wg = wg_ref[...]
    h2d = 2 * hidden
    h0s = [jnp.zeros((n, hidden), jnp.float32)] * k_batch
    h1ss = [jnp.zeros((n, hidden), jnp.float32)] * k_batch
    xs = [[None] * t_len for _ in range(k_batch)]
    for t in range(t_len + 1):
        for j in range(k_batch):
            cat = jnp.concatenate([h0s[j], h1ss[j], ones_col], axis=1)
            m = jnp.dot(cat, wg, preferred_element_type=jnp.float32)
            if t >= 1:
                rz1 = jax.nn.sigmoid(m[:, 3 * hidden:5 * hidden]
                                     + m[:, 6 * hidden:8 * hidden])
                ng1 = jnp.tanh(m[:, 5 * hidden:6 * hidden]
                               + rz1[:, :hidden] * m[:, 8 * hidden:])
                h1ss[j] = ng1 + rz1[:, hidden:] * (h1ss[j] - ng1)
                xs[j][t - 1] = h1ss[j]
            if t < t_len:
                gi = gi0[j * gn + t * n:j * gn + (t + 1) * n]
                rz = jax.nn.sigmoid(gi[:, :h2d] + m[:, :h2d])
                ng = jnp.tanh(gi[:, h2d:] + rz[:, :hidden] * m[:, h2d:3 * hidden])
                h0s[j] = ng + rz[:, hidden:] * (h0s[j] - ng)

    zrow = jnp.zeros((1, t_len * hidden), jnp.float32)
    prows = []
    for j in range(k_batch):
        xr = jnp.concatenate(xs[j], axis=1)
        pdn = jnp.concatenate([zrow, xr[:-1]], axis=0)
        pup = jnp.concatenate([xr[1:], zrow], axis=0)
        prows.append(jnp.concatenate([pdn, xr, pup], axis=1))
    patches = jnp.concatenate(prows, axis=0)

    conv = jnp.dot(patches, wm_ref[...],
                   preferred_element_type=jnp.float32) + cb_ref[...]
    conv = jnp.where(conv > 0, conv, a * conv)
    h = jnp.dot(conv, w1c_ref[...],
                preferred_element_type=jnp.float32) + b1c_ref[...]
    h = jnp.where(h > 0, h, a * h)
    res = jnp.dot(h, w2c_ref[...],
                  preferred_element_type=jnp.float32) + b2c_ref[...]
    out_ref[...] = res.reshape(k_batch, n, -1)


def kernel(x, adj, gat1_w, gat1_asrc, gat1_adst, gat1_bias,
           gat2_w, gat2_asrc, gat2_adst, gat2_bias, prelu_a,
           gru_wih0_t, gru_whh0_t, gru_bih0, gru_bhh0,
           gru_wih1_t, gru_whh1_t, gru_bih1, gru_bhh1,
           conv_w, conv_b, out1_w_t, out1_b, out2_w_t, out2_b):
    b, t, n, fin = x.shape
    gn = t * n
    hidden = gat2_w.shape[1]
    num_heads = gat1_w.shape[1] // hidden
    pred = conv_w.shape[0]
    out_f = out2_w_t.shape[1]

    x_flat = x.reshape(b, gn, fin)

    eye_n = jnp.eye(n, dtype=jnp.float32)
    allow = jnp.maximum(adj, eye_n)
    bm = jnp.where(allow > 0, 0.0, -1e30).astype(jnp.float32)
    bmp = bm.reshape(b, t // 2, 2, n, n).transpose(0, 1, 3, 2, 4)
    bmp = bmp.reshape(b, t // 2, n, 2 * n)

    asd1 = jnp.concatenate([gat1_asrc, gat1_adst], axis=0).T
    asd2 = jnp.concatenate([gat2_asrc, gat2_adst], axis=0).T

    rn = jnp.arange(gn)
    p512 = jnp.eye(gn, dtype=jnp.float32)[(rn % n) * t + rn // n]

    zh = jnp.zeros((hidden, 3 * hidden), jnp.float32)
    wg = jnp.concatenate([
        jnp.concatenate([gru_whh0_t, gru_wih1_t, zh], axis=1),
        jnp.concatenate([zh, zh, gru_whh1_t], axis=1),
        jnp.concatenate([gru_bhh0, gru_bih1, gru_bhh1], axis=1),
    ], axis=0)

    hh_idx = jnp.arange(hidden)
    band = hh_idx[:, None] - hh_idx[None, :]
    sel = jnp.stack([(band == dx - 1).astype(jnp.float32)
                     for dx in range(3)])
    wm = jnp.einsum('cdye,eab->ydacb', conv_w, sel).reshape(
        3 * t * hidden, pred * hidden)
    cb = jnp.repeat(conv_b[:, 0], hidden)[None, :]

    eye_c = jnp.eye(pred, dtype=jnp.float32)
    w1c = jnp.kron(eye_c, out1_w_t)
    b1c = jnp.tile(out1_b, (1, pred))
    w2c = jnp.kron(eye_c, out2_w_t)
    b2c = jnp.tile(out2_b, (1, pred))

    k_batch = 8
    kern = functools.partial(_fused_kernel, t_len=t, n_nodes=n,
                             hidden=hidden, k_batch=k_batch)
    rep = lambda i: (0, 0)
    out = pl.pallas_call(
        kern,
        out_shape=jax.ShapeDtypeStruct((b, n, pred * out_f), jnp.float32),
        grid=(b // k_batch,),
        in_specs=[
            pl.BlockSpec(memory_space=pltpu.MemorySpace.SMEM),
            pl.BlockSpec((k_batch, gn, fin), lambda i: (i, 0, 0)),
            pl.BlockSpec((k_batch, t // 2, n, 2 * n),
                         lambda i: (i, 0, 0, 0)),
            pl.BlockSpec((gn, gn), rep),
            pl.BlockSpec((fin, num_heads * hidden), rep),
            pl.BlockSpec((num_heads * hidden, 2), rep),
            pl.BlockSpec((1, num_heads * hidden), rep),
            pl.BlockSpec((num_heads * hidden, hidden), rep),
            pl.BlockSpec((hidden, 2), rep),
            pl.BlockSpec((1, hidden), rep),
            pl.BlockSpec((hidden, 3 * hidden), rep),
            pl.BlockSpec((1, 3 * hidden), rep),
            pl.BlockSpec((2 * hidden + 1, 9 * hidden), rep),
            pl.BlockSpec((3 * t * hidden, pred * hidden), rep),
            pl.BlockSpec((1, pred * hidden), rep),
            pl.BlockSpec((pred * hidden, pred * hidden), rep),
            pl.BlockSpec((1, pred * hidden), rep),
            pl.BlockSpec((pred * hidden, pred * out_f), rep),
            pl.BlockSpec((1, pred * out_f), rep),
        ],
        out_specs=pl.BlockSpec((k_batch, n, pred * out_f),
                               lambda i: (i, 0, 0)),
        compiler_params=pltpu.CompilerParams(
            dimension_semantics=("parallel",)),
    )(prelu_a, x_flat, bmp, p512,
      gat1_w, asd1, gat1_bias,
      gat2_w, asd2, gat2_bias,
      gru_wih0_t, gru_bih0, wg,
      wm, cb, w1c, b1c, w2c, b2c)

    return out.reshape(b, n, pred, out_f).transpose(0, 2, 1, 3)


# speedup vs baseline: 1.3536x; 1.3212x over previous
import functools

import jax
import jax.numpy as jnp
from jax import lax
from jax.experimental import pallas as pl
from jax.experimental.pallas import tpu as pltpu


def _fused_kernel(alpha_ref, x_ref, bm_ref, p_ref,
                  w1_ref, asd1_ref, b1_ref,
                  w2_ref, asd2_ref, b2_ref,
                  wih0_ref, bih0_ref, wg_ref,
                  wm_ref, cb_ref, w1c_ref, b1c_ref, w2c_ref, b2c_ref,
                  out_ref, *, t_len, n_nodes, hidden, k_batch):
    a = alpha_ref[0, 0]
    n = n_nodes
    gn = t_len * n
    npair = t_len // 2

    sel2 = (lax.broadcasted_iota(jnp.int32, (2, 2 * n), 1) // n
            == lax.broadcasted_iota(jnp.int32, (2, 2 * n), 0)
            ).astype(jnp.float32)
    ones_col = jnp.ones((n, 1), jnp.float32)

    def gat_layer(h_in, w, asd_w, bias):
        h = jnp.dot(h_in, w, preferred_element_type=jnp.float32)
        asd = jnp.dot(h, asd_w, preferred_element_type=jnp.float32)
        a_dT = jnp.transpose(asd)
        outs = []
        for j in range(k_batch):
            es = []
            for p in range(npair):
                base = j * gn + p * 2 * n
                a_s3 = jnp.concatenate(
                    [asd[base:base + n, 0:1],
                     asd[base + n:base + 2 * n, 0:1], ones_col], axis=1)
                m3 = jnp.concatenate(
                    [sel2, a_dT[1:2, base:base + 2 * n]], axis=0)
                e = jnp.dot(a_s3, m3,
                            preferred_element_type=jnp.float32)
                e = jnp.where(e > 0, e, 0.2 * e)
                es.append(e + bm_ref[j, p])
            mx = [jnp.max(e, axis=0, keepdims=True) for e in es]
            prs = [jnp.exp(e - m) for e, m in zip(es, mx)]
            dens = [jnp.sum(pr, axis=0, keepdims=True) for pr in prs]
            atts = [pr * pl.reciprocal(den, approx=True)
                    for pr, den in zip(prs, dens)]
            for p in range(npair):
                base = j * gn + p * 2 * n
                outs.append(lax.dot_general(
                    atts[p][:, :n], h[base:base + n],
                    (((0,), (0,)), ((), ())),
                    preferred_element_type=jnp.float32))
                outs.append(lax.dot_general(
                    atts[p][:, n:], h[base + n:base + 2 * n],
                    (((0,), (0,)), ((), ())),
                    preferred_element_type=jnp.float32))
        o = jnp.concatenate(outs, axis=0) + bias
        return jnp.where(o > 0, o, a * o)

    x = x_ref[...].reshape(k_batch * gn, -1)
    h1 = gat_layer(x, w1_ref[...], asd1_ref[...], b1_ref[...])
    h2 = gat_layer(h1, w2_ref[...], asd2_ref[...], b2_ref[...])

    h2p = jnp.concatenate(
        [jnp.dot(p_ref[...], h2[j * gn:(j + 1) * gn],
                 preferred_element_type=jnp.float32)
         for j in range(k_batch)], axis=0)
    gi0 = jnp.dot(h2p, wih0_ref[...],
                  preferred_element_type=jnp.float32) + bih0_ref[...]

    wg = wg_ref[...]
    h2d = 2 * hidden
    h0s = [jnp.zeros((n, hidden), jnp.float32)] * k_batch
    h1ss = [jnp.zeros((n, hidden), jnp.float32)] * k_batch
    xs = [[None] * t_len for _ in range(k_batch)]
    for t in range(t_len + 1):
        ms = [jnp.dot(jnp.concatenate([h0s[j], h1ss[j], ones_col], axis=1),
                      wg, preferred_element_type=jnp.float32)
              for j in range(k_batch)]
        for j in range(k_batch):
            m = ms[j]
            if t >= 1:
                rz1 = jax.nn.sigmoid(m[:, 3 * hidden:5 * hidden]
                                     + m[:, 6 * hidden:8 * hidden])
                ng1 = jnp.tanh(m[:, 5 * hidden:6 * hidden]
                               + rz1[:, :hidden] * m[:, 8 * hidden:])
                h1ss[j] = ng1 + rz1[:, hidden:] * (h1ss[j] - ng1)
                xs[j][t - 1] = h1ss[j]
            if t < t_len:
                gi = gi0[j * gn + t * n:j * gn + (t + 1) * n]
                rz = jax.nn.sigmoid(gi[:, :h2d] + m[:, :h2d])
                ng = jnp.tanh(gi[:, h2d:] + rz[:, :hidden] * m[:, h2d:3 * hidden])
                h0s[j] = ng + rz[:, hidden:] * (h0s[j] - ng)

    zrow = jnp.zeros((1, t_len * hidden), jnp.float32)
    prows = []
    for j in range(k_batch):
        xr = jnp.concatenate(xs[j], axis=1)
        pdn = jnp.concatenate([zrow, xr[:-1]], axis=0)
        pup = jnp.concatenate([xr[1:], zrow], axis=0)
        prows.append(jnp.concatenate([pdn, xr, pup], axis=1))
    patches = jnp.concatenate(prows, axis=0)

    conv = jnp.dot(patches, wm_ref[...],
                   preferred_element_type=jnp.float32) + cb_ref[...]
    conv = jnp.where(conv > 0, conv, a * conv)
    h = jnp.dot(conv, w1c_ref[...],
                preferred_element_type=jnp.float32) + b1c_ref[...]
    h = jnp.where(h > 0, h, a * h)
    res = jnp.dot(h, w2c_ref[...],
                  preferred_element_type=jnp.float32) + b2c_ref[...]
    out_ref[...] = res.reshape(k_batch, n, -1)


def kernel(x, adj, gat1_w, gat1_asrc, gat1_adst, gat1_bias,
           gat2_w, gat2_asrc, gat2_adst, gat2_bias, prelu_a,
           gru_wih0_t, gru_whh0_t, gru_bih0, gru_bhh0,
           gru_wih1_t, gru_whh1_t, gru_bih1, gru_bhh1,
           conv_w, conv_b, out1_w_t, out1_b, out2_w_t, out2_b):
    b, t, n, fin = x.shape
    gn = t * n
    hidden = gat2_w.shape[1]
    num_heads = gat1_w.shape[1] // hidden
    pred = conv_w.shape[0]
    out_f = out2_w_t.shape[1]

    x_flat = x.reshape(b, gn, fin)

    eye_n = jnp.eye(n, dtype=jnp.float32)
    allow = jnp.maximum(adj, eye_n)
    bm = jnp.where(allow > 0, 0.0, -1e30).astype(jnp.float32)
    bmp = bm.reshape(b, t // 2, 2, n, n).transpose(0, 1, 3, 2, 4)
    bmp = bmp.reshape(b, t // 2, n, 2 * n)

    asd1 = jnp.concatenate([gat1_asrc, gat1_adst], axis=0).T
    asd2 = jnp.concatenate([gat2_asrc, gat2_adst], axis=0).T

    rn = jnp.arange(gn)
    p512 = jnp.eye(gn, dtype=jnp.float32)[(rn % n) * t + rn // n]

    zh = jnp.zeros((hidden, 3 * hidden), jnp.float32)
    wg = jnp.concatenate([
        jnp.concatenate([gru_whh0_t, gru_wih1_t, zh], axis=1),
        jnp.concatenate([zh, zh, gru_whh1_t], axis=1),
        jnp.concatenate([gru_bhh0, gru_bih1, gru_bhh1], axis=1),
    ], axis=0)

    hh_idx = jnp.arange(hidden)
    band = hh_idx[:, None] - hh_idx[None, :]
    sel = jnp.stack([(band == dx - 1).astype(jnp.float32)
                     for dx in range(3)])
    wm = jnp.einsum('cdye,eab->ydacb', conv_w, sel).reshape(
        3 * t * hidden, pred * hidden)
    cb = jnp.repeat(conv_b[:, 0], hidden)[None, :]

    eye_c = jnp.eye(pred, dtype=jnp.float32)
    w1c = jnp.kron(eye_c, out1_w_t)
    b1c = jnp.tile(out1_b, (1, pred))
    w2c = jnp.kron(eye_c, out2_w_t)
    b2c = jnp.tile(out2_b, (1, pred))

    k_batch = 8
    kern = functools.partial(_fused_kernel, t_len=t, n_nodes=n,
                             hidden=hidden, k_batch=k_batch)
    rep = lambda i: (0, 0)
    out = pl.pallas_call(
        kern,
        out_shape=jax.ShapeDtypeStruct((b, n, pred * out_f), jnp.float32),
        grid=(b // k_batch,),
        in_specs=[
            pl.BlockSpec(memory_space=pltpu.MemorySpace.SMEM),
            pl.BlockSpec((k_batch, gn, fin), lambda i: (i, 0, 0)),
            pl.BlockSpec((k_batch, t // 2, n, 2 * n),
                         lambda i: (i, 0, 0, 0)),
            pl.BlockSpec((gn, gn), rep),
            pl.BlockSpec((fin, num_heads * hidden), rep),
            pl.BlockSpec((num_heads * hidden, 2), rep),
            pl.BlockSpec((1, num_heads * hidden), rep),
            pl.BlockSpec((num_heads * hidden, hidden), rep),
            pl.BlockSpec((hidden, 2), rep),
            pl.BlockSpec((1, hidden), rep),
            pl.BlockSpec((hidden, 3 * hidden), rep),
            pl.BlockSpec((1, 3 * hidden), rep),
            pl.BlockSpec((2 * hidden + 1, 9 * hidden), rep),
            pl.BlockSpec((3 * t * hidden, pred * hidden), rep),
            pl.BlockSpec((1, pred * hidden), rep),
            pl.BlockSpec((pred * hidden, pred * hidden), rep),
            pl.BlockSpec((1, pred * hidden), rep),
            pl.BlockSpec((pred * hidden, pred * out_f), rep),
            pl.BlockSpec((1, pred * out_f), rep),
        ],
        out_specs=pl.BlockSpec((k_batch, n, pred * out_f),
                               lambda i: (i, 0, 0)),
        compiler_params=pltpu.CompilerParams(
            dimension_semantics=("parallel",)),
    )(prelu_a, x_flat, bmp, p512,
      gat1_w, asd1, gat1_bias,
      gat2_w, asd2, gat2_bias,
      gru_wih0_t, gru_bih0, wg,
      wm, cb, w1c, b1c, w2c, b2c)

    return out.reshape(b, n, pred, out_f).transpose(0, 2, 1, 3)


# speedup vs baseline: 1.3549x; 1.0010x over previous
import functools

import jax
import jax.numpy as jnp
from jax import lax
from jax.experimental import pallas as pl
from jax.experimental.pallas import tpu as pltpu


def _fused_kernel(alpha_ref, x_ref, bm_ref, p_ref,
                  w1_ref, asd1_ref, b1_ref,
                  w2_ref, asd2_ref, b2_ref,
                  wih0_ref, bih0_ref, wg_ref,
                  wm_ref, cb_ref, w1c_ref, b1c_ref, w2c_ref, b2c_ref,
                  out_ref, *, t_len, n_nodes, hidden, k_batch):
    a = alpha_ref[0, 0]
    n = n_nodes
    gn = t_len * n
    npair = t_len // 2

    sel2 = (lax.broadcasted_iota(jnp.int32, (2, 2 * n), 1) // n
            == lax.broadcasted_iota(jnp.int32, (2, 2 * n), 0)
            ).astype(jnp.float32)
    ones_col = jnp.ones((n, 1), jnp.float32)

    def gat_layer(h_in, w, asd_w, bias):
        h = jnp.dot(h_in, w, preferred_element_type=jnp.float32)
        asd = jnp.dot(h, asd_w, preferred_element_type=jnp.float32)
        a_dT = jnp.transpose(asd)
        outs = []
        for j in range(k_batch):
            es = []
            for p in range(npair):
                base = j * gn + p * 2 * n
                a_s3 = jnp.concatenate(
                    [asd[base:base + n, 0:1],
                     asd[base + n:base + 2 * n, 0:1], ones_col], axis=1)
                m3 = jnp.concatenate(
                    [sel2, a_dT[1:2, base:base + 2 * n]], axis=0)
                e = jnp.dot(a_s3, m3,
                            preferred_element_type=jnp.float32)
                e = jnp.where(e > 0, e, 0.2 * e)
                es.append(e + bm_ref[j, p])
            mx = [jnp.max(e, axis=0, keepdims=True) for e in es]
            prs = [jnp.exp(e - m) for e, m in zip(es, mx)]
            dens = [jnp.sum(pr, axis=0, keepdims=True) for pr in prs]
            atts = [pr * pl.reciprocal(den, approx=True)
                    for pr, den in zip(prs, dens)]
            for p in range(npair):
                base = j * gn + p * 2 * n
                outs.append(lax.dot_general(
                    atts[p][:, :n], h[base:base + n],
                    (((0,), (0,)), ((), ())),
                    preferred_element_type=jnp.float32))
                outs.append(lax.dot_general(
                    atts[p][:, n:], h[base + n:base + 2 * n],
                    (((0,), (0,)), ((), ())),
                    preferred_element_type=jnp.float32))
        o = jnp.concatenate(outs, axis=0) + bias
        return jnp.where(o > 0, o, a * o)

    x = x_ref[...].reshape(k_batch * gn, -1)
    h1 = gat_layer(x, w1_ref[...], asd1_ref[...], b1_ref[...])
    h2 = gat_layer(h1, w2_ref[...], asd2_ref[...], b2_ref[...])

    h2p = jnp.concatenate(
        [jnp.dot(p_ref[...], h2[j * gn:(j + 1) * gn],
                 preferred_element_type=jnp.float32)
         for j in range(k_batch)], axis=0)
    gi0 = jnp.dot(h2p, wih0_ref[...],
                  preferred_element_type=jnp.float32) + bih0_ref[...]

    wg = wg_ref[...]
    h2d = 2 * hidden
    h0s = [jnp.zeros((n, hidden), jnp.float32)] * k_batch
    h1ss = [jnp.zeros((n, hidden), jnp.float32)] * k_batch
    xs = [[None] * t_len for _ in range(k_batch)]
    for t in range(t_len + 1):
        ms = [jnp.dot(jnp.concatenate([h0s[j], h1ss[j], ones_col], axis=1),
                      wg, preferred_element_type=jnp.float32)
              for j in range(k_batch)]
        if t >= 1:
            rz1s = [jax.nn.sigmoid(m[:, 3 * hidden:5 * hidden]
                                   + m[:, 6 * hidden:8 * hidden])
                    for m in ms]
            ng1s = [jnp.tanh(m[:, 5 * hidden:6 * hidden]
                             + rz1[:, :hidden] * m[:, 8 * hidden:])
                    for m, rz1 in zip(ms, rz1s)]
            for j in range(k_batch):
                h1ss[j] = ng1s[j] + rz1s[j][:, hidden:] * (h1ss[j] - ng1s[j])
                xs[j][t - 1] = h1ss[j]
        if t < t_len:
            gis = [gi0[j * gn + t * n:j * gn + (t + 1) * n]
                   for j in range(k_batch)]
            rzs = [jax.nn.sigmoid(gi[:, :h2d] + m[:, :h2d])
                   for gi, m in zip(gis, ms)]
            ngs = [jnp.tanh(gi[:, h2d:] + rz[:, :hidden] * m[:, h2d:3 * hidden])
                   for gi, rz, m in zip(gis, rzs, ms)]
            for j in range(k_batch):
                h0s[j] = ngs[j] + rzs[j][:, hidden:] * (h0s[j] - ngs[j])

    zrow = jnp.zeros((1, t_len * hidden), jnp.float32)
    prows = []
    for j in range(k_batch):
        xr = jnp.concatenate(xs[j], axis=1)
        pdn = jnp.concatenate([zrow, xr[:-1]], axis=0)
        pup = jnp.concatenate([xr[1:], zrow], axis=0)
        prows.append(jnp.concatenate([pdn, xr, pup], axis=1))
    patches = jnp.concatenate(prows, axis=0)

    conv = jnp.dot(patches, wm_ref[...],
                   preferred_element_type=jnp.float32) + cb_ref[...]
    conv = jnp.where(conv > 0, conv, a * conv)
    h = jnp.dot(conv, w1c_ref[...],
                preferred_element_type=jnp.float32) + b1c_ref[...]
    h = jnp.where(h > 0, h, a * h)
    res = jnp.dot(h, w2c_ref[...],
                  preferred_element_type=jnp.float32) + b2c_ref[...]
    out_ref[...] = res.reshape(k_batch, n, -1)


def kernel(x, adj, gat1_w, gat1_asrc, gat1_adst, gat1_bias,
           gat2_w, gat2_asrc, gat2_adst, gat2_bias, prelu_a,
           gru_wih0_t, gru_whh0_t, gru_bih0, gru_bhh0,
           gru_wih1_t, gru_whh1_t, gru_bih1, gru_bhh1,
           conv_w, conv_b, out1_w_t, out1_b, out2_w_t, out2_b):
    b, t, n, fin = x.shape
    gn = t * n
    hidden = gat2_w.shape[1]
    num_heads = gat1_w.shape[1] // hidden
    pred = conv_w.shape[0]
    out_f = out2_w_t.shape[1]

    x_flat = x.reshape(b, gn, fin)

    eye_n = jnp.eye(n, dtype=jnp.float32)
    allow = jnp.maximum(adj, eye_n)
    bm = jnp.where(allow > 0, 0.0, -1e30).astype(jnp.float32)
    bmp = bm.reshape(b, t // 2, 2, n, n).transpose(0, 1, 3, 2, 4)
    bmp = bmp.reshape(b, t // 2, n, 2 * n)

    asd1 = jnp.concatenate([gat1_asrc, gat1_adst], axis=0).T
    asd2 = jnp.concatenate([gat2_asrc, gat2_adst], axis=0).T

    rn = jnp.arange(gn)
    p512 = jnp.eye(gn, dtype=jnp.float32)[(rn % n) * t + rn // n]

    zh = jnp.zeros((hidden, 3 * hidden), jnp.float32)
    wg = jnp.concatenate([
        jnp.concatenate([gru_whh0_t, gru_wih1_t, zh], axis=1),
        jnp.concatenate([zh, zh, gru_whh1_t], axis=1),
        jnp.concatenate([gru_bhh0, gru_bih1, gru_bhh1], axis=1),
    ], axis=0)

    hh_idx = jnp.arange(hidden)
    band = hh_idx[:, None] - hh_idx[None, :]
    sel = jnp.stack([(band == dx - 1).astype(jnp.float32)
                     for dx in range(3)])
    wm = jnp.einsum('cdye,eab->ydacb', conv_w, sel).reshape(
        3 * t * hidden, pred * hidden)
    cb = jnp.repeat(conv_b[:, 0], hidden)[None, :]

    eye_c = jnp.eye(pred, dtype=jnp.float32)
    w1c = jnp.kron(eye_c, out1_w_t)
    b1c = jnp.tile(out1_b, (1, pred))
    w2c = jnp.kron(eye_c, out2_w_t)
    b2c = jnp.tile(out2_b, (1, pred))

    k_batch = 8
    kern = functools.partial(_fused_kernel, t_len=t, n_nodes=n,
                             hidden=hidden, k_batch=k_batch)
    rep = lambda i: (0, 0)
    out = pl.pallas_call(
        kern,
        out_shape=jax.ShapeDtypeStruct((b, n, pred * out_f), jnp.float32),
        grid=(b // k_batch,),
        in_specs=[
            pl.BlockSpec(memory_space=pltpu.MemorySpace.SMEM),
            pl.BlockSpec((k_batch, gn, fin), lambda i: (i, 0, 0)),
            pl.BlockSpec((k_batch, t // 2, n, 2 * n),
                         lambda i: (i, 0, 0, 0)),
            pl.BlockSpec((gn, gn), rep),
            pl.BlockSpec((fin, num_heads * hidden), rep),
            pl.BlockSpec((num_heads * hidden, 2), rep),
            pl.BlockSpec((1, num_heads * hidden), rep),
            pl.BlockSpec((num_heads * hidden, hidden), rep),
            pl.BlockSpec((hidden, 2), rep),
            pl.BlockSpec((1, hidden), rep),
            pl.BlockSpec((hidden, 3 * hidden), rep),
            pl.BlockSpec((1, 3 * hidden), rep),
            pl.BlockSpec((2 * hidden + 1, 9 * hidden), rep),
            pl.BlockSpec((3 * t * hidden, pred * hidden), rep),
            pl.BlockSpec((1, pred * hidden), rep),
            pl.BlockSpec((pred * hidden, pred * hidden), rep),
            pl.BlockSpec((1, pred * hidden), rep),
            pl.BlockSpec((pred * hidden, pred * out_f), rep),
            pl.BlockSpec((1, pred * out_f), rep),
        ],
        out_specs=pl.BlockSpec((k_batch, n, pred * out_f),
                               lambda i: (i, 0, 0)),
        compiler_params=pltpu.CompilerParams(
            dimension_semantics=("parallel",)),
    )(prelu_a, x_flat, bmp, p512,
      gat1_w, asd1, gat1_bias,
      gat2_w, asd2, gat2_bias,
      gru_wih0_t, gru_bih0, wg,
      wm, cb, w1c, b1c, w2c, b2c)

    return out.reshape(b, n, pred, out_f).transpose(0, 2, 1, 3)
